# baseline (speedup 1.0000x reference)
import functools

import jax
import jax.numpy as jnp
from jax import lax
from jax.experimental import pallas as pl
from jax.experimental.pallas import tpu as pltpu
from jax.experimental.pallas import tpu_sc as plsc

_B = 8
_BSC = 4
_N = 2048
_LANES = 16
_BL = 32
_NSC = 16
_QBLK = 8
_WPB = 8
_QPW = _N // _WPB
_CHUNKS = _N // _BL
_QB_TC = 256

_MESH = plsc.VectorSubcoreMesh(core_axis_name="c", subcore_axis_name="s")


@functools.partial(
    pl.kernel,
    out_type=jax.ShapeDtypeStruct((2 * _NSC, _LANES), jnp.float32),
    mesh=_MESH,
    compiler_params=pltpu.CompilerParams(use_tc_tiling_on_sc=False,
                                         needs_layout_passes=False),
    scratch_types=[
        pltpu.VMEM((3, _N), jnp.int32),
        pltpu.VMEM((3, _N), jnp.bfloat16),
        pltpu.VMEM((_N,), jnp.bfloat16),
        pltpu.VMEM(((_WPB - 1) * _N,), jnp.bfloat16),
        pltpu.VMEM((_LANES,), jnp.float32),
        pltpu.VMEM_SHARED((_NSC * _N,), jnp.bfloat16),
    ],
)
def _chamfer_sc(a1_hbm, a2_hbm, out_hbm, q_v, t_v, c_v, nb_v, acc_v,
                shared_v):
    cid = lax.axis_index("c")
    sid = lax.axis_index("s")
    w = cid * _NSC + sid
    b = w // _WPB
    r = w % _WPB

    pltpu.sync_copy(a2_hbm.at[b], q_v)
    pltpu.sync_copy(a1_hbm.at[b], t_v)

    inf_b = jnp.full((_BL,), jnp.inf, jnp.bfloat16)
    lane = lax.iota(jnp.int32, _LANES)
    perms = [lax.bitwise_xor(lane, jnp.int32(1 << k)) for k in range(4)]

    def _hmin(v):
        for p in perms:
            v = jnp.minimum(v, v.at[p].get(mode="promise_in_bounds"))
        return v

    def _hsum(v):
        for p in perms:
            v = v + v.at[p].get(mode="promise_in_bounds")
        return v

    def _splat_bf(x):
        return plsc.bitcast(jnp.full((_LANES,), x), jnp.bfloat16)

    def _unpack_f32(v):
        bits = plsc.bitcast(v, jnp.int32)
        hi = plsc.bitcast(jnp.bitwise_and(bits, jnp.int32(-65536)),
                          jnp.float32)
        lo = plsc.bitcast(lax.shift_left(bits, 16), jnp.float32)
        return lo, hi

    def init_body(i, carry):
        c_v[pl.ds(i * _BL, _BL)] = inf_b
        return carry

    lax.fori_loop(0, _CHUNKS, init_body, 0)

    qbase = r * _QPW

    def qblock_body(qb, acc):
        qoff = qbase + qb * _LANES
        qxv = q_v[0, pl.ds(qoff, _LANES)]
        qyv = q_v[1, pl.ds(qoff, _LANES)]
        qzv = q_v[2, pl.ds(qoff, _LANES)]
        for half in range(2):
            qx = [_splat_bf(qxv[half * _QBLK + u]) for u in range(_QBLK)]
            qy = [_splat_bf(qyv[half * _QBLK + u]) for u in range(_QBLK)]
            qz = [_splat_bf(qzv[half * _QBLK + u]) for u in range(_QBLK)]

            def chunk_body(tt, mins):
                sl = pl.ds(tt * _BL, _BL)
                txv = t_v[0, sl]
                tyv = t_v[1, sl]
                tzv = t_v[2, sl]
                cv = c_v[sl]
                out = []
                for u in range(_QBLK):
                    dx = txv - qx[u]
                    dy = tyv - qy[u]
                    dz = tzv - qz[u]
                    d = dx * dx + dy * dy + dz * dz
                    out.append(jnp.minimum(mins[u], d))
                    cv = jnp.minimum(cv, d)
                c_v[sl] = cv
                return tuple(out)

            mins = lax.fori_loop(0, _CHUNKS, chunk_body, (inf_b,) * _QBLK)
            for u in range(_QBLK):
                ma, mb = _unpack_f32(mins[u])
                acc = acc + _hmin(jnp.minimum(ma, mb))
        return acc

    acc = lax.fori_loop(0, _QPW // _LANES, qblock_body,
                        jnp.zeros((_LANES,), jnp.float32))

    pltpu.sync_copy(c_v, shared_v.at[pl.ds(sid * _N, _N)])
    plsc.subcore_barrier()

    @pl.when(r == 0)
    def _():
        for k in range(_WPB - 1):
            pltpu.sync_copy(shared_v.at[pl.ds((sid + 1 + k) * _N, _N)],
                            nb_v.at[pl.ds(k * _N, _N)])

        def merge_body(i, csum):
            t0 = i * _BL
            cm = c_v[pl.ds(t0, _BL)]
            for k in range(_WPB - 1):
                cm = jnp.minimum(cm, nb_v[pl.ds(k * _N + t0, _BL)])
            ca, cb = _unpack_f32(cm)
            return csum + (ca + cb)

        csum = lax.fori_loop(0, _CHUNKS, merge_body,
                             jnp.zeros((_LANES,), jnp.float32))
        acc_v[...] = acc + _hsum(csum)

    @pl.when(r != 0)
    def _():
        acc_v[...] = acc

    pltpu.sync_copy(acc_v, out_hbm.at[w])


def _tc_body(a1_ref, a2_ref, row_ref, col_ref, cmin_ref):
    bb = pl.program_id(0)
    j = pl.program_id(1)
    tm = a1_ref[0]
    qm = a2_ref[0]
    dot = lax.dot_general(qm, tm, (((0,), (0,)), ((), ())),
                          precision=lax.Precision.HIGHEST,
                          preferred_element_type=jnp.float32)
    q2 = jnp.sum(qm * qm, axis=0)[:, None]
    t2 = jnp.sum(tm * tm, axis=0)[None, :]
    dist = (q2 + t2) - (dot + dot)
    row_ref[bb, j] = jnp.sum(jnp.min(dist, axis=1))
    bmin = jnp.min(dist, axis=0)[None, :]

    @pl.when(j == 0)
    def _():
        cmin_ref[...] = bmin

    @pl.when(j != 0)
    def _():
        cmin_ref[...] = jnp.minimum(cmin_ref[...], bmin)

    col_ref[bb, 0] = jnp.sum(cmin_ref[...])


_chamfer_tc = pl.pallas_call(
    _tc_body,
    grid=(_B - _BSC, _N // _QB_TC),
    in_specs=[
        pl.BlockSpec((1, 3, _N), lambda b, j: (b, 0, 0)),
        pl.BlockSpec((1, 3, _QB_TC), lambda b, j: (b, 0, j)),
    ],
    out_specs=[
        pl.BlockSpec((_B - _BSC, _N // _QB_TC), lambda b, j: (0, 0),
                     memory_space=pltpu.SMEM),
        pl.BlockSpec((_B - _BSC, 1), lambda b, j: (0, 0),
                     memory_space=pltpu.SMEM),
    ],
    out_shape=[
        jax.ShapeDtypeStruct((_B - _BSC, _N // _QB_TC), jnp.float32),
        jax.ShapeDtypeStruct((_B - _BSC, 1), jnp.float32),
    ],
    scratch_shapes=[pltpu.VMEM((1, _N), jnp.float32)],
)


def kernel(array1, array2):
    a1t = jnp.transpose(array1, (0, 2, 1))
    a2t = jnp.transpose(array2, (0, 2, 1))
    a1b = a1t[:_BSC].astype(jnp.bfloat16)
    a2b = a2t[:_BSC].astype(jnp.bfloat16)
    qbits = lax.bitcast_convert_type(a2b, jnp.uint16).astype(jnp.uint32)
    a2p = (qbits | (qbits << jnp.uint32(16))).astype(jnp.int32)
    sc_partials = _chamfer_sc(a1b, a2p)
    rowsums, colsums = _chamfer_tc(a1t[_BSC:], a2t[_BSC:])
    weight = jnp.float32(100.0 * 0.5 / (_B * _N))
    total = jnp.sum(sc_partials[:, 0]) + jnp.sum(rowsums) + jnp.sum(colsums)
    return weight * total


# speedup vs baseline: 1.6504x; 1.6504x over previous
import functools

import jax
import jax.numpy as jnp
from jax import lax
from jax.experimental import pallas as pl
from jax.experimental.pallas import tpu as pltpu
from jax.experimental.pallas import tpu_sc as plsc

_B = 8
_BSC = 4
_N = 2048
_LANES = 16
_BL = 32
_NSC = 16
_QBLK = 8
_WPB = 8
_QPW = _N // _WPB
_CHUNKS = _N // _BL
_QB_TC = 512

_MESH = plsc.VectorSubcoreMesh(core_axis_name="c", subcore_axis_name="s")


@functools.partial(
    pl.kernel,
    out_type=jax.ShapeDtypeStruct((2 * _NSC, _LANES), jnp.float32),
    mesh=_MESH,
    compiler_params=pltpu.CompilerParams(use_tc_tiling_on_sc=False,
                                         needs_layout_passes=False),
    scratch_types=[
        pltpu.VMEM((3, _N), jnp.int32),
        pltpu.VMEM((3, _N), jnp.bfloat16),
        pltpu.VMEM((_N,), jnp.bfloat16),
        pltpu.VMEM(((_WPB - 1) * _N,), jnp.bfloat16),
        pltpu.VMEM((_LANES,), jnp.float32),
        pltpu.VMEM_SHARED((_NSC * _N,), jnp.bfloat16),
    ],
)
def _chamfer_sc(a1_hbm, a2_hbm, out_hbm, q_v, t_v, c_v, nb_v, acc_v,
                shared_v):
    cid = lax.axis_index("c")
    sid = lax.axis_index("s")
    w = cid * _NSC + sid
    b = w // _WPB
    r = w % _WPB

    pltpu.sync_copy(a2_hbm.at[b], q_v)
    pltpu.sync_copy(a1_hbm.at[b], t_v)

    inf_b = jnp.full((_BL,), jnp.inf, jnp.bfloat16)
    lane = lax.iota(jnp.int32, _LANES)
    perms = [lax.bitwise_xor(lane, jnp.int32(1 << k)) for k in range(4)]

    def _hmin(v):
        for p in perms:
            v = jnp.minimum(v, v.at[p].get(mode="promise_in_bounds"))
        return v

    def _hsum(v):
        for p in perms:
            v = v + v.at[p].get(mode="promise_in_bounds")
        return v

    def _splat_bf(x):
        return plsc.bitcast(jnp.full((_LANES,), x), jnp.bfloat16)

    def _unpack_f32(v):
        bits = plsc.bitcast(v, jnp.int32)
        hi = plsc.bitcast(jnp.bitwise_and(bits, jnp.int32(-65536)),
                          jnp.float32)
        lo = plsc.bitcast(lax.shift_left(bits, 16), jnp.float32)
        return lo, hi

    def init_body(i, carry):
        c_v[pl.ds(i * _BL, _BL)] = inf_b
        return carry

    lax.fori_loop(0, _CHUNKS, init_body, 0)

    qbase = r * _QPW

    def qblock_body(qb, acc):
        qoff = qbase + qb * _LANES
        qxv = q_v[0, pl.ds(qoff, _LANES)]
        qyv = q_v[1, pl.ds(qoff, _LANES)]
        qzv = q_v[2, pl.ds(qoff, _LANES)]
        for half in range(2):
            qx = [_splat_bf(qxv[half * _QBLK + u]) for u in range(_QBLK)]
            qy = [_splat_bf(qyv[half * _QBLK + u]) for u in range(_QBLK)]
            qz = [_splat_bf(qzv[half * _QBLK + u]) for u in range(_QBLK)]

            def chunk_body(tt, mins):
                sl = pl.ds(tt * _BL, _BL)
                txv = t_v[0, sl]
                tyv = t_v[1, sl]
                tzv = t_v[2, sl]
                cv = c_v[sl]
                out = []
                for u in range(_QBLK):
                    dx = txv - qx[u]
                    dy = tyv - qy[u]
                    dz = tzv - qz[u]
                    d = dx * dx + dy * dy + dz * dz
                    out.append(jnp.minimum(mins[u], d))
                    cv = jnp.minimum(cv, d)
                c_v[sl] = cv
                return tuple(out)

            mins = lax.fori_loop(0, _CHUNKS, chunk_body, (inf_b,) * _QBLK)
            for u in range(_QBLK):
                ma, mb = _unpack_f32(mins[u])
                acc = acc + _hmin(jnp.minimum(ma, mb))
        return acc

    acc = lax.fori_loop(0, _QPW // _LANES, qblock_body,
                        jnp.zeros((_LANES,), jnp.float32))

    pltpu.sync_copy(c_v, shared_v.at[pl.ds(sid * _N, _N)])
    plsc.subcore_barrier()

    @pl.when(r == 0)
    def _():
        for k in range(_WPB - 1):
            pltpu.sync_copy(shared_v.at[pl.ds((sid + 1 + k) * _N, _N)],
                            nb_v.at[pl.ds(k * _N, _N)])

        def merge_body(i, csum):
            t0 = i * _BL
            cm = c_v[pl.ds(t0, _BL)]
            for k in range(_WPB - 1):
                cm = jnp.minimum(cm, nb_v[pl.ds(k * _N + t0, _BL)])
            ca, cb = _unpack_f32(cm)
            return csum + (ca + cb)

        csum = lax.fori_loop(0, _CHUNKS, merge_body,
                             jnp.zeros((_LANES,), jnp.float32))
        acc_v[...] = acc + _hsum(csum)

    @pl.when(r != 0)
    def _():
        acc_v[...] = acc

    pltpu.sync_copy(acc_v, out_hbm.at[w])


def _tc_body(a1_ref, a2_ref, row_ref, col_ref, cmin_ref):
    bb = pl.program_id(0)
    j = pl.program_id(1)
    tx = a1_ref[0, 0, :][None, :]
    ty = a1_ref[0, 1, :][None, :]
    tz = a1_ref[0, 2, :][None, :]
    qx = a2_ref[0, 0, :][:, None]
    qy = a2_ref[0, 1, :][:, None]
    qz = a2_ref[0, 2, :][:, None]
    dx = qx - tx
    dy = qy - ty
    dz = qz - tz
    dist = dx * dx + dy * dy + dz * dz
    row_ref[bb, j] = jnp.sum(jnp.min(dist, axis=1))
    bmin = jnp.min(dist, axis=0)[None, :]

    @pl.when(j == 0)
    def _():
        cmin_ref[...] = bmin

    @pl.when(j != 0)
    def _():
        cmin_ref[...] = jnp.minimum(cmin_ref[...], bmin)

    col_ref[bb, 0] = jnp.sum(cmin_ref[...])


_chamfer_tc = pl.pallas_call(
    _tc_body,
    grid=(_B - _BSC, _N // _QB_TC),
    in_specs=[
        pl.BlockSpec((1, 3, _N), lambda b, j: (b, 0, 0)),
        pl.BlockSpec((1, 3, _QB_TC), lambda b, j: (b, 0, j)),
    ],
    out_specs=[
        pl.BlockSpec((_B - _BSC, _N // _QB_TC), lambda b, j: (0, 0),
                     memory_space=pltpu.SMEM),
        pl.BlockSpec((_B - _BSC, 1), lambda b, j: (0, 0),
                     memory_space=pltpu.SMEM),
    ],
    out_shape=[
        jax.ShapeDtypeStruct((_B - _BSC, _N // _QB_TC), jnp.float32),
        jax.ShapeDtypeStruct((_B - _BSC, 1), jnp.float32),
    ],
    scratch_shapes=[pltpu.VMEM((1, _N), jnp.float32)],
)


def kernel(array1, array2):
    a1t = jnp.transpose(array1, (0, 2, 1))
    a2t = jnp.transpose(array2, (0, 2, 1))
    a1b = a1t[:_BSC].astype(jnp.bfloat16)
    a2b = a2t[:_BSC].astype(jnp.bfloat16)
    qbits = lax.bitcast_convert_type(a2b, jnp.uint16).astype(jnp.uint32)
    a2p = (qbits | (qbits << jnp.uint32(16))).astype(jnp.int32)
    sc_partials = _chamfer_sc(a1b, a2p)
    rowsums, colsums = _chamfer_tc(a1t[_BSC:], a2t[_BSC:])
    weight = jnp.float32(100.0 * 0.5 / (_B * _N))
    total = jnp.sum(sc_partials[:, 0]) + jnp.sum(rowsums) + jnp.sum(colsums)
    return weight * total


# speedup vs baseline: 1.6515x; 1.0007x over previous
import functools

import jax
import jax.numpy as jnp
from jax import lax
from jax.experimental import pallas as pl
from jax.experimental.pallas import tpu as pltpu
from jax.experimental.pallas import tpu_sc as plsc

_B = 8
_BSC = 4
_N = 2048
_LANES = 16
_BL = 32
_NSC = 16
_QBLK = 8
_WPB = 8
_QPW = _N // _WPB
_CHUNKS = _N // _BL
_QB_TC = 1024

_MESH = plsc.VectorSubcoreMesh(core_axis_name="c", subcore_axis_name="s")


@functools.partial(
    pl.kernel,
    out_type=jax.ShapeDtypeStruct((2 * _NSC, _LANES), jnp.float32),
    mesh=_MESH,
    compiler_params=pltpu.CompilerParams(use_tc_tiling_on_sc=False,
                                         needs_layout_passes=False),
    scratch_types=[
        pltpu.VMEM((3, _N), jnp.int32),
        pltpu.VMEM((3, _N), jnp.bfloat16),
        pltpu.VMEM((_N,), jnp.bfloat16),
        pltpu.VMEM(((_WPB - 1) * _N,), jnp.bfloat16),
        pltpu.VMEM((_LANES,), jnp.float32),
        pltpu.VMEM_SHARED((_NSC * _N,), jnp.bfloat16),
    ],
)
def _chamfer_sc(a1_hbm, a2_hbm, out_hbm, q_v, t_v, c_v, nb_v, acc_v,
                shared_v):
    cid = lax.axis_index("c")
    sid = lax.axis_index("s")
    w = cid * _NSC + sid
    b = w // _WPB
    r = w % _WPB

    pltpu.sync_copy(a2_hbm.at[b], q_v)
    pltpu.sync_copy(a1_hbm.at[b], t_v)

    inf_b = jnp.full((_BL,), jnp.inf, jnp.bfloat16)
    lane = lax.iota(jnp.int32, _LANES)
    perms = [lax.bitwise_xor(lane, jnp.int32(1 << k)) for k in range(4)]

    def _hmin(v):
        for p in perms:
            v = jnp.minimum(v, v.at[p].get(mode="promise_in_bounds"))
        return v

    def _hsum(v):
        for p in perms:
            v = v + v.at[p].get(mode="promise_in_bounds")
        return v

    def _splat_bf(x):
        return plsc.bitcast(jnp.full((_LANES,), x), jnp.bfloat16)

    def _unpack_f32(v):
        bits = plsc.bitcast(v, jnp.int32)
        hi = plsc.bitcast(jnp.bitwise_and(bits, jnp.int32(-65536)),
                          jnp.float32)
        lo = plsc.bitcast(lax.shift_left(bits, 16), jnp.float32)
        return lo, hi

    def init_body(i, carry):
        c_v[pl.ds(i * _BL, _BL)] = inf_b
        return carry

    lax.fori_loop(0, _CHUNKS, init_body, 0)

    qbase = r * _QPW

    def qblock_body(qb, acc):
        qoff = qbase + qb * _LANES
        qxv = q_v[0, pl.ds(qoff, _LANES)]
        qyv = q_v[1, pl.ds(qoff, _LANES)]
        qzv = q_v[2, pl.ds(qoff, _LANES)]
        for half in range(2):
            qx = [_splat_bf(qxv[half * _QBLK + u]) for u in range(_QBLK)]
            qy = [_splat_bf(qyv[half * _QBLK + u]) for u in range(_QBLK)]
            qz = [_splat_bf(qzv[half * _QBLK + u]) for u in range(_QBLK)]

            def chunk_body(tt, mins):
                sl = pl.ds(tt * _BL, _BL)
                txv = t_v[0, sl]
                tyv = t_v[1, sl]
                tzv = t_v[2, sl]
                cv = c_v[sl]
                out = []
                for u in range(_QBLK):
                    dx = txv - qx[u]
                    dy = tyv - qy[u]
                    dz = tzv - qz[u]
                    d = dx * dx + dy * dy + dz * dz
                    out.append(jnp.minimum(mins[u], d))
                    cv = jnp.minimum(cv, d)
                c_v[sl] = cv
                return tuple(out)

            mins = lax.fori_loop(0, _CHUNKS, chunk_body, (inf_b,) * _QBLK)
            for u in range(_QBLK):
                ma, mb = _unpack_f32(mins[u])
                acc = acc + _hmin(jnp.minimum(ma, mb))
        return acc

    acc = lax.fori_loop(0, _QPW // _LANES, qblock_body,
                        jnp.zeros((_LANES,), jnp.float32))

    pltpu.sync_copy(c_v, shared_v.at[pl.ds(sid * _N, _N)])
    plsc.subcore_barrier()

    @pl.when(r == 0)
    def _():
        for k in range(_WPB - 1):
            pltpu.sync_copy(shared_v.at[pl.ds((sid + 1 + k) * _N, _N)],
                            nb_v.at[pl.ds(k * _N, _N)])

        def merge_body(i, csum):
            t0 = i * _BL
            cm = c_v[pl.ds(t0, _BL)]
            for k in range(_WPB - 1):
                cm = jnp.minimum(cm, nb_v[pl.ds(k * _N + t0, _BL)])
            ca, cb = _unpack_f32(cm)
            return csum + (ca + cb)

        csum = lax.fori_loop(0, _CHUNKS, merge_body,
                             jnp.zeros((_LANES,), jnp.float32))
        acc_v[...] = acc + _hsum(csum)

    @pl.when(r != 0)
    def _():
        acc_v[...] = acc

    pltpu.sync_copy(acc_v, out_hbm.at[w])


def _tc_body(a1_ref, a2_ref, row_ref, col_ref, cmin_ref):
    bb = pl.program_id(0)
    j = pl.program_id(1)
    tx = a1_ref[0, 0, :][None, :]
    ty = a1_ref[0, 1, :][None, :]
    tz = a1_ref[0, 2, :][None, :]
    qx = a2_ref[0, 0, :][:, None]
    qy = a2_ref[0, 1, :][:, None]
    qz = a2_ref[0, 2, :][:, None]
    dx = qx - tx
    dy = qy - ty
    dz = qz - tz
    dist = dx * dx + dy * dy + dz * dz
    row_ref[bb, j] = jnp.sum(jnp.min(dist, axis=1))
    bmin = jnp.min(dist, axis=0)[None, :]

    @pl.when(j == 0)
    def _():
        cmin_ref[...] = bmin

    @pl.when(j != 0)
    def _():
        cmin_ref[...] = jnp.minimum(cmin_ref[...], bmin)

    col_ref[bb, 0] = jnp.sum(cmin_ref[...])


_chamfer_tc = pl.pallas_call(
    _tc_body,
    grid=(_B - _BSC, _N // _QB_TC),
    in_specs=[
        pl.BlockSpec((1, 3, _N), lambda b, j: (b, 0, 0)),
        pl.BlockSpec((1, 3, _QB_TC), lambda b, j: (b, 0, j)),
    ],
    out_specs=[
        pl.BlockSpec((_B - _BSC, _N // _QB_TC), lambda b, j: (0, 0),
                     memory_space=pltpu.SMEM),
        pl.BlockSpec((_B - _BSC, 1), lambda b, j: (0, 0),
                     memory_space=pltpu.SMEM),
    ],
    out_shape=[
        jax.ShapeDtypeStruct((_B - _BSC, _N // _QB_TC), jnp.float32),
        jax.ShapeDtypeStruct((_B - _BSC, 1), jnp.float32),
    ],
    scratch_shapes=[pltpu.VMEM((1, _N), jnp.float32)],
)


def kernel(array1, array2):
    a1t = jnp.transpose(array1, (0, 2, 1))
    a2t = jnp.transpose(array2, (0, 2, 1))
    a1b = a1t[:_BSC].astype(jnp.bfloat16)
    a2b = a2t[:_BSC].astype(jnp.bfloat16)
    qbits = lax.bitcast_convert_type(a2b, jnp.uint16).astype(jnp.uint32)
    a2p = (qbits | (qbits << jnp.uint32(16))).astype(jnp.int32)
    sc_partials = _chamfer_sc(a1b, a2p)
    rowsums, colsums = _chamfer_tc(a1t[_BSC:], a2t[_BSC:])
    weight = jnp.float32(100.0 * 0.5 / (_B * _N))
    total = jnp.sum(sc_partials[:, 0]) + jnp.sum(rowsums) + jnp.sum(colsums)
    return weight * total
